# baseline (speedup 1.0000x reference)
import functools

import jax
import jax.numpy as jnp
from jax.experimental import pallas as pl
from jax.experimental.pallas import tpu as pltpu

_BN = 2048


def _chamfer_body(x_ref, yt_ref, out_ref, ryb_ref, colmin_ref, *, n_blocks, m):
    i = pl.program_id(1)

    @pl.when(i == 0)
    def _init():
        out_ref[...] = jnp.zeros_like(out_ref)
        colmin_ref[...] = jnp.full_like(colmin_ref, 1e30)
        y = yt_ref[0]
        ry = (y[0:1, :] * y[0:1, :] + y[1:2, :] * y[1:2, :]
              + y[2:3, :] * y[2:3, :])
        ryb_ref[...] = jnp.broadcast_to(ry, (8, m))

    x = x_ref[0]
    zz2 = jnp.dot(x * (-2.0), yt_ref[0],
                  preferred_element_type=jnp.float32)
    rx = jnp.sum(x * x, axis=1, keepdims=True)
    rs = None
    for g in range(_BN // 8):
        sl = slice(g * 8, (g + 1) * 8)
        rxg = jnp.broadcast_to(rx[sl, :], (8, m))
        p = (rxg + ryb_ref[...]) + zz2[sl, :]
        colmin_ref[...] = jnp.minimum(colmin_ref[...], p)
        rmin = jnp.min(p, axis=1, keepdims=True)
        rs = rmin if rs is None else rs + rmin
    out_ref[...] += jnp.sum(rs)

    @pl.when(i == n_blocks - 1)
    def _fin():
        cm = jnp.min(colmin_ref[...], axis=0)
        out_ref[...] += jnp.sum(cm)


def kernel(preds, gts):
    b, n, _ = gts.shape
    _, m, _ = preds.shape
    yt = jnp.transpose(preds.astype(jnp.float32), (0, 2, 1))
    n_blocks = n // _BN
    out = pl.pallas_call(
        functools.partial(_chamfer_body, n_blocks=n_blocks, m=m),
        grid=(b, n_blocks),
        in_specs=[
            pl.BlockSpec((1, _BN, 3), lambda bi, i: (bi, i, 0)),
            pl.BlockSpec((1, 3, m), lambda bi, i: (bi, 0, 0)),
        ],
        out_specs=pl.BlockSpec((1, 8, 128), lambda bi, i: (bi, 0, 0)),
        out_shape=jax.ShapeDtypeStruct((b, 8, 128), jnp.float32),
        scratch_shapes=[
            pltpu.VMEM((8, m), jnp.float32),
            pltpu.VMEM((8, m), jnp.float32),
        ],
        compiler_params=pltpu.CompilerParams(
            dimension_semantics=("parallel", "arbitrary"),
        ),
        name="chamfer_loss",
    )(gts.astype(jnp.float32), yt)
    return jnp.sum(out[:, 0, 0])


# speedup vs baseline: 1.0217x; 1.0217x over previous
import functools

import jax
import jax.numpy as jnp
from jax.experimental import pallas as pl
from jax.experimental.pallas import tpu as pltpu

_BN = 2048


def _chamfer_body(x_ref, yt_ref, out_ref, ryb_ref, colmin_ref, *, n_blocks, m):
    i = pl.program_id(1)

    @pl.when(i == 0)
    def _init():
        out_ref[...] = jnp.zeros_like(out_ref)
        colmin_ref[...] = jnp.full_like(colmin_ref, 1e30)
        y = yt_ref[0]
        ry = (y[0:1, :] * y[0:1, :] + y[1:2, :] * y[1:2, :]
              + y[2:3, :] * y[2:3, :])
        ryb_ref[...] = jnp.broadcast_to(ry, (8, m))

    x = x_ref[0]
    xs = x * (-2.0)
    rx = jnp.sum(x * x, axis=1, keepdims=True)
    mc = 256
    zz2c = [jnp.dot(xs, yt_ref[0][:, c * mc:(c + 1) * mc],
                    preferred_element_type=jnp.float32)
            for c in range(m // mc)]
    rs = None
    for g in range(_BN // 8):
        sl = slice(g * 8, (g + 1) * 8)
        rxg = jnp.broadcast_to(rx[sl, :], (8, mc))
        rmin = None
        for c in range(m // mc):
            cs = slice(c * mc, (c + 1) * mc)
            p = (rxg + ryb_ref[:, cs]) + zz2c[c][sl, :]
            colmin_ref[:, cs] = jnp.minimum(colmin_ref[:, cs], p)
            pm = jnp.min(p, axis=1, keepdims=True)
            rmin = pm if rmin is None else jnp.minimum(rmin, pm)
        rs = rmin if rs is None else rs + rmin
    out_ref[...] += jnp.sum(rs)

    @pl.when(i == n_blocks - 1)
    def _fin():
        cm = jnp.min(colmin_ref[...], axis=0)
        out_ref[...] += jnp.sum(cm)


def kernel(preds, gts):
    b, n, _ = gts.shape
    _, m, _ = preds.shape
    yt = jnp.transpose(preds.astype(jnp.float32), (0, 2, 1))
    n_blocks = n // _BN
    out = pl.pallas_call(
        functools.partial(_chamfer_body, n_blocks=n_blocks, m=m),
        grid=(b, n_blocks),
        in_specs=[
            pl.BlockSpec((1, _BN, 3), lambda bi, i: (bi, i, 0)),
            pl.BlockSpec((1, 3, m), lambda bi, i: (bi, 0, 0)),
        ],
        out_specs=pl.BlockSpec((1, 8, 128), lambda bi, i: (bi, 0, 0)),
        out_shape=jax.ShapeDtypeStruct((b, 8, 128), jnp.float32),
        scratch_shapes=[
            pltpu.VMEM((8, m), jnp.float32),
            pltpu.VMEM((8, m), jnp.float32),
        ],
        compiler_params=pltpu.CompilerParams(
            dimension_semantics=("parallel", "arbitrary"),
        ),
        name="chamfer_loss",
    )(gts.astype(jnp.float32), yt)
    return jnp.sum(out[:, 0, 0])


# speedup vs baseline: 1.0582x; 1.0357x over previous
import functools

import jax
import jax.numpy as jnp
from jax.experimental import pallas as pl
from jax.experimental.pallas import tpu as pltpu

_BN = 2048


def _chamfer_body(x_ref, yt_ref, out_ref, ryb_ref, colmin_ref, *, n_blocks, m):
    i = pl.program_id(1)

    @pl.when(i == 0)
    def _init():
        out_ref[...] = jnp.zeros_like(out_ref)
        colmin_ref[...] = jnp.full_like(colmin_ref, 1e30)
        y = yt_ref[0]
        ry = (y[0:1, :] * y[0:1, :] + y[1:2, :] * y[1:2, :]
              + y[2:3, :] * y[2:3, :])
        ryb_ref[...] = jnp.broadcast_to(ry, (8, m))

    x = x_ref[0]
    xs = x * (-2.0)
    rx = jnp.sum(x * x, axis=1, keepdims=True)
    mc = 256
    zz2c = [jnp.dot(xs, yt_ref[0][:, c * mc:(c + 1) * mc],
                    preferred_element_type=jnp.float32)
            for c in range(m // mc)]
    rs = None
    for g in range(_BN // 8):
        sl = slice(g * 8, (g + 1) * 8)
        rxg = jnp.broadcast_to(rx[sl, :], (8, mc))
        racc = None
        for c in range(m // mc):
            cs = slice(c * mc, (c + 1) * mc)
            p = (rxg + ryb_ref[:, cs]) + zz2c[c][sl, :]
            colmin_ref[:, cs] = jnp.minimum(colmin_ref[:, cs], p)
            pm = jnp.minimum(p[:, :128], p[:, 128:])
            racc = pm if racc is None else jnp.minimum(racc, pm)
        rmin = jnp.min(racc, axis=1, keepdims=True)
        rs = rmin if rs is None else rs + rmin
    out_ref[...] += jnp.sum(rs)

    @pl.when(i == n_blocks - 1)
    def _fin():
        cm = jnp.min(colmin_ref[...], axis=0)
        out_ref[...] += jnp.sum(cm)


def kernel(preds, gts):
    b, n, _ = gts.shape
    _, m, _ = preds.shape
    yt = jnp.transpose(preds.astype(jnp.float32), (0, 2, 1))
    n_blocks = n // _BN
    out = pl.pallas_call(
        functools.partial(_chamfer_body, n_blocks=n_blocks, m=m),
        grid=(b, n_blocks),
        in_specs=[
            pl.BlockSpec((1, _BN, 3), lambda bi, i: (bi, i, 0)),
            pl.BlockSpec((1, 3, m), lambda bi, i: (bi, 0, 0)),
        ],
        out_specs=pl.BlockSpec((1, 8, 128), lambda bi, i: (bi, 0, 0)),
        out_shape=jax.ShapeDtypeStruct((b, 8, 128), jnp.float32),
        scratch_shapes=[
            pltpu.VMEM((8, m), jnp.float32),
            pltpu.VMEM((8, m), jnp.float32),
        ],
        compiler_params=pltpu.CompilerParams(
            dimension_semantics=("parallel", "arbitrary"),
        ),
        name="chamfer_loss",
    )(gts.astype(jnp.float32), yt)
    return jnp.sum(out[:, 0, 0])


# speedup vs baseline: 1.1594x; 1.0956x over previous
import functools

import jax
import jax.numpy as jnp
from jax.experimental import pallas as pl
from jax.experimental.pallas import tpu as pltpu

_BN = 2048


def _chamfer_body(x_ref, yt_ref, out_ref, ryb_ref, colmin_ref, *, n_blocks, m):
    i = pl.program_id(1)

    @pl.when(i == 0)
    def _init():
        out_ref[...] = jnp.zeros_like(out_ref)
        colmin_ref[...] = jnp.full_like(colmin_ref, 1e30)
        y = yt_ref[0]
        ry = (y[0:1, :] * y[0:1, :] + y[1:2, :] * y[1:2, :]
              + y[2:3, :] * y[2:3, :])
        ryb_ref[...] = jnp.broadcast_to(ry, (8, m))

    x = x_ref[0]
    xs = x * (-2.0)
    rx = jnp.sum(x * x, axis=1, keepdims=True)
    mc = 256
    gb = 128
    rs = None
    for g0 in range(0, _BN, gb):
        xg = xs[g0:g0 + gb, :]
        rxbc = [jnp.broadcast_to(rx[g0 + k * 8:g0 + (k + 1) * 8, :], (8, mc))
                for k in range(gb // 8)]
        racc = [None] * (gb // 8)
        for c in range(m // mc):
            cs = slice(c * mc, (c + 1) * mc)
            z = jnp.dot(xg, yt_ref[0][:, cs],
                        preferred_element_type=jnp.float32)
            ryc = ryb_ref[:, cs]
            cm = colmin_ref[:, cs]
            for k in range(gb // 8):
                p = (rxbc[k] + ryc) + z[k * 8:(k + 1) * 8, :]
                cm = jnp.minimum(cm, p)
                pm = jnp.minimum(p[:, :128], p[:, 128:])
                racc[k] = pm if racc[k] is None else jnp.minimum(racc[k], pm)
            colmin_ref[:, cs] = cm
        for k in range(gb // 8):
            rmin = jnp.min(racc[k], axis=1, keepdims=True)
            rs = rmin if rs is None else rs + rmin
    out_ref[...] += jnp.sum(rs)

    @pl.when(i == n_blocks - 1)
    def _fin():
        cm = jnp.min(colmin_ref[...], axis=0)
        out_ref[...] += jnp.sum(cm)


def kernel(preds, gts):
    b, n, _ = gts.shape
    _, m, _ = preds.shape
    yt = jnp.transpose(preds.astype(jnp.float32), (0, 2, 1))
    n_blocks = n // _BN
    out = pl.pallas_call(
        functools.partial(_chamfer_body, n_blocks=n_blocks, m=m),
        grid=(b, n_blocks),
        in_specs=[
            pl.BlockSpec((1, _BN, 3), lambda bi, i: (bi, i, 0)),
            pl.BlockSpec((1, 3, m), lambda bi, i: (bi, 0, 0)),
        ],
        out_specs=pl.BlockSpec((1, 8, 128), lambda bi, i: (bi, 0, 0)),
        out_shape=jax.ShapeDtypeStruct((b, 8, 128), jnp.float32),
        scratch_shapes=[
            pltpu.VMEM((8, m), jnp.float32),
            pltpu.VMEM((8, m), jnp.float32),
        ],
        compiler_params=pltpu.CompilerParams(
            dimension_semantics=("parallel", "arbitrary"),
        ),
        name="chamfer_loss",
    )(gts.astype(jnp.float32), yt)
    return jnp.sum(out[:, 0, 0])


# speedup vs baseline: 1.1984x; 1.0336x over previous
import functools

import jax
import jax.numpy as jnp
from jax.experimental import pallas as pl
from jax.experimental.pallas import tpu as pltpu

_BN = 4096


def _chamfer_body(x_ref, yt_ref, out_ref, ryb_ref, colmin_ref, *, n_blocks, m):
    i = pl.program_id(1)

    @pl.when(i == 0)
    def _init():
        out_ref[...] = jnp.zeros_like(out_ref)
        colmin_ref[...] = jnp.full_like(colmin_ref, 1e30)
        y = yt_ref[0]
        ry = (y[0:1, :] * y[0:1, :] + y[1:2, :] * y[1:2, :]
              + y[2:3, :] * y[2:3, :])
        ryb_ref[...] = jnp.broadcast_to(ry, (8, m))

    x = x_ref[0]
    xs = x * (-2.0)
    rx = jnp.sum(x * x, axis=1, keepdims=True)
    mc = 256
    gb = 128
    rs = None
    for g0 in range(0, _BN, gb):
        xg = xs[g0:g0 + gb, :]
        rxbc = [jnp.broadcast_to(rx[g0 + k * 8:g0 + (k + 1) * 8, :], (8, mc))
                for k in range(gb // 8)]
        racc = [None] * (gb // 8)
        for c in range(m // mc):
            cs = slice(c * mc, (c + 1) * mc)
            z = jnp.dot(xg, yt_ref[0][:, cs],
                        preferred_element_type=jnp.float32)
            ryc = ryb_ref[:, cs]
            cm = colmin_ref[:, cs]
            for k in range(gb // 8):
                p = (rxbc[k] + ryc) + z[k * 8:(k + 1) * 8, :]
                cm = jnp.minimum(cm, p)
                pm = jnp.minimum(p[:, :128], p[:, 128:])
                racc[k] = pm if racc[k] is None else jnp.minimum(racc[k], pm)
            colmin_ref[:, cs] = cm
        for k in range(gb // 8):
            rmin = jnp.min(racc[k], axis=1, keepdims=True)
            rs = rmin if rs is None else rs + rmin
    out_ref[...] += jnp.sum(rs)

    @pl.when(i == n_blocks - 1)
    def _fin():
        cm = jnp.min(colmin_ref[...], axis=0)
        out_ref[...] += jnp.sum(cm)


def kernel(preds, gts):
    b, n, _ = gts.shape
    _, m, _ = preds.shape
    yt = jnp.transpose(preds.astype(jnp.float32), (0, 2, 1))
    n_blocks = n // _BN
    out = pl.pallas_call(
        functools.partial(_chamfer_body, n_blocks=n_blocks, m=m),
        grid=(b, n_blocks),
        in_specs=[
            pl.BlockSpec((1, _BN, 3), lambda bi, i: (bi, i, 0)),
            pl.BlockSpec((1, 3, m), lambda bi, i: (bi, 0, 0)),
        ],
        out_specs=pl.BlockSpec((1, 8, 128), lambda bi, i: (bi, 0, 0)),
        out_shape=jax.ShapeDtypeStruct((b, 8, 128), jnp.float32),
        scratch_shapes=[
            pltpu.VMEM((8, m), jnp.float32),
            pltpu.VMEM((8, m), jnp.float32),
        ],
        compiler_params=pltpu.CompilerParams(
            dimension_semantics=("parallel", "arbitrary"),
        ),
        name="chamfer_loss",
    )(gts.astype(jnp.float32), yt)
    return jnp.sum(out[:, 0, 0])


# speedup vs baseline: 1.2019x; 1.0029x over previous
import functools

import jax
import jax.numpy as jnp
from jax.experimental import pallas as pl
from jax.experimental.pallas import tpu as pltpu

_BN = 4096


def _chamfer_body(x_ref, yt_ref, out_ref, ryb_ref, colmin_ref, *, n_blocks, m):
    i = pl.program_id(1)

    @pl.when(i == 0)
    def _init():
        out_ref[...] = jnp.zeros_like(out_ref)
        colmin_ref[...] = jnp.full_like(colmin_ref, 1e30)
        y = yt_ref[0]
        ry = (y[0:1, :] * y[0:1, :] + y[1:2, :] * y[1:2, :]
              + y[2:3, :] * y[2:3, :])
        ryb_ref[...] = jnp.broadcast_to(ry, (8, m))

    x = x_ref[0]
    xs = x * (-2.0)
    rx = jnp.sum(x * x, axis=1, keepdims=True)
    mc = 256
    gb = 256
    rs = None
    for g0 in range(0, _BN, gb):
        xg = xs[g0:g0 + gb, :]
        rxbc = [jnp.broadcast_to(rx[g0 + k * 8:g0 + (k + 1) * 8, :], (8, mc))
                for k in range(gb // 8)]
        racc = [None] * (gb // 8)
        for c in range(m // mc):
            cs = slice(c * mc, (c + 1) * mc)
            z = jnp.dot(xg, yt_ref[0][:, cs],
                        preferred_element_type=jnp.float32)
            ryc = ryb_ref[:, cs]
            cm = colmin_ref[:, cs]
            for k in range(gb // 8):
                p = (rxbc[k] + ryc) + z[k * 8:(k + 1) * 8, :]
                cm = jnp.minimum(cm, p)
                pm = jnp.minimum(p[:, :128], p[:, 128:])
                racc[k] = pm if racc[k] is None else jnp.minimum(racc[k], pm)
            colmin_ref[:, cs] = cm
        for k in range(gb // 8):
            rmin = jnp.min(racc[k], axis=1, keepdims=True)
            rs = rmin if rs is None else rs + rmin
    out_ref[...] += jnp.sum(rs)

    @pl.when(i == n_blocks - 1)
    def _fin():
        cm = jnp.min(colmin_ref[...], axis=0)
        out_ref[...] += jnp.sum(cm)


def kernel(preds, gts):
    b, n, _ = gts.shape
    _, m, _ = preds.shape
    yt = jnp.transpose(preds.astype(jnp.float32), (0, 2, 1))
    n_blocks = n // _BN
    out = pl.pallas_call(
        functools.partial(_chamfer_body, n_blocks=n_blocks, m=m),
        grid=(b, n_blocks),
        in_specs=[
            pl.BlockSpec((1, _BN, 3), lambda bi, i: (bi, i, 0)),
            pl.BlockSpec((1, 3, m), lambda bi, i: (bi, 0, 0)),
        ],
        out_specs=pl.BlockSpec((1, 8, 128), lambda bi, i: (bi, 0, 0)),
        out_shape=jax.ShapeDtypeStruct((b, 8, 128), jnp.float32),
        scratch_shapes=[
            pltpu.VMEM((8, m), jnp.float32),
            pltpu.VMEM((8, m), jnp.float32),
        ],
        compiler_params=pltpu.CompilerParams(
            dimension_semantics=("parallel", "arbitrary"),
        ),
        name="chamfer_loss",
    )(gts.astype(jnp.float32), yt)
    return jnp.sum(out[:, 0, 0])
